# Initial kernel scaffold; baseline (speedup 1.0000x reference)
import jax
import jax.numpy as jnp
from jax.experimental import pallas as pl


def kernel(x, enc_W1, enc_b1, enc_W2, enc_b2, g1_W, g1_b, g2_W, g2_b, ep_W1, ep_b1, ep_W2, ep_b2, edge_index):
    raise NotImplementedError("write your pallas kernel here")


# speedup vs baseline: 4.3617x; 4.3617x over previous
import functools

import jax
import jax.numpy as jnp
from jax import lax
from jax.experimental import pallas as pl
from jax.experimental.pallas import tpu as pltpu
from jax.experimental.pallas import tpu_sc as plsc

N = 10000
E = 160000
H = 64
NV = 128
NC, NS = 2, 16
NW = NC * NS
EPW = E // NW
CH = 125
NCH = EPW // CH
RPT = N // NS
DW = 16

f32 = jnp.float32


def _sc_mesh():
    return plsc.VectorSubcoreMesh(
        core_axis_name="c", subcore_axis_name="s", num_cores=NC, num_subcores=NS
    )


def _deg_body(srcr, zeros16, out, idx_v, ones_v, acc):
    c = lax.axis_index("c")
    s = lax.axis_index("s")
    wid = c * NS + s
    pltpu.sync_copy(srcr.at[wid], idx_v)
    pltpu.sync_copy(zeros16.at[s], acc.at[pl.ds(s * RPT, RPT)])

    def fill(i, carry):
        ones_v[i] = jnp.full((16,), 0.0625, f32)
        return carry

    lax.fori_loop(0, CH, fill, 0)
    plsc.subcore_barrier()

    def chunk(j, carry):
        pltpu.sync_copy(ones_v, acc.at[idx_v.at[j]], add=True)
        return carry

    lax.fori_loop(0, NCH, chunk, 0)
    plsc.subcore_barrier()
    pltpu.sync_copy(acc.at[pl.ds(s * RPT, RPT)], out.at[c, s])


@functools.cache
def _deg_call():
    return pl.kernel(
        _deg_body,
        out_type=jax.ShapeDtypeStruct((NC, NS, RPT, DW), f32),
        mesh=_sc_mesh(),
        compiler_params=pltpu.CompilerParams(use_tc_tiling_on_sc=False),
        scratch_types=[
            pltpu.VMEM((NCH, CH), jnp.int32),
            pltpu.VMEM((CH, DW), f32),
            pltpu.VMEM_SHARED((N, DW), f32),
        ],
    )


def _msg_body(srcr, dstr, zs, zeros64, out, src_v, dst_v, rows_v, acc):
    c = lax.axis_index("c")
    s = lax.axis_index("s")
    wid = c * NS + s
    pltpu.sync_copy(srcr.at[wid], src_v)
    pltpu.sync_copy(dstr.at[wid], dst_v)
    pltpu.sync_copy(zeros64.at[s], acc.at[pl.ds(s * RPT, RPT)])
    plsc.subcore_barrier()

    def chunk(j, carry):
        pltpu.sync_copy(zs.at[dst_v.at[j]], rows_v)
        pltpu.sync_copy(rows_v, acc.at[src_v.at[j]], add=True)
        return carry

    lax.fori_loop(0, NCH, chunk, 0)
    plsc.subcore_barrier()
    pltpu.sync_copy(acc.at[pl.ds(s * RPT, RPT)], out.at[c, s])


@functools.cache
def _msg_call():
    return pl.kernel(
        _msg_body,
        out_type=jax.ShapeDtypeStruct((NC, NS, RPT, H), f32),
        mesh=_sc_mesh(),
        compiler_params=pltpu.CompilerParams(use_tc_tiling_on_sc=False),
        scratch_types=[
            pltpu.VMEM((NCH, CH), jnp.int32),
            pltpu.VMEM((NCH, CH), jnp.int32),
            pltpu.VMEM((CH, H), f32),
            pltpu.VMEM_SHARED((N, H), f32),
        ],
    )


def _edge_body(srcr, dstr, a1, a2, g1o, g2o, src_v, dst_v, r1_v, r2_v):
    c = lax.axis_index("c")
    s = lax.axis_index("s")
    wid = c * NS + s
    pltpu.sync_copy(srcr.at[wid], src_v)
    pltpu.sync_copy(dstr.at[wid], dst_v)

    def chunk(j, carry):
        pltpu.sync_copy(a1.at[src_v.at[j]], r1_v)
        pltpu.sync_copy(r1_v, g1o.at[wid, j])
        pltpu.sync_copy(a2.at[dst_v.at[j]], r2_v)
        pltpu.sync_copy(r2_v, g2o.at[wid, j])
        return carry

    lax.fori_loop(0, NCH, chunk, 0)


@functools.cache
def _edge_call():
    return pl.kernel(
        _edge_body,
        out_type=(
            jax.ShapeDtypeStruct((NW, NCH, CH, H), f32),
            jax.ShapeDtypeStruct((NW, NCH, CH, H), f32),
        ),
        mesh=_sc_mesh(),
        compiler_params=pltpu.CompilerParams(use_tc_tiling_on_sc=False),
        scratch_types=[
            pltpu.VMEM((NCH, CH), jnp.int32),
            pltpu.VMEM((NCH, CH), jnp.int32),
            pltpu.VMEM((CH, H), f32),
            pltpu.VMEM((CH, H), f32),
        ],
    )


BN = 1000


def _dis(d0, d1):
    deg = 1.0 + jnp.sum(d0[...] + d1[...], axis=1, keepdims=True)
    return lax.rsqrt(deg)


def _enc_body(x_ref, w1, b1, w2, b2, gw, gb, d0, d1, zs_ref):
    ne = jnp.maximum(x_ref[...] @ w1[...] + b1[...], 0.0) @ w2[...] + b2[...]
    y = ne @ gw[...] + gb[...]
    zs_ref[...] = _dis(d0, d1) * y


def _mid_body(zs, p0, p1, d0, d1, gw, gb, out):
    dis = _dis(d0, d1)
    h = jnp.maximum(dis * (zs[...] + p0[...] + p1[...]), 0.0)
    out[...] = dis * (h @ gw[...] + gb[...])


def _fin_body(zs, p0, p1, d0, d1, wa, wb, b1, hout, a1out, a2out):
    dis = _dis(d0, d1)
    h = dis * (zs[...] + p0[...] + p1[...])
    hout[...] = h
    a1out[...] = h @ wa[...] + b1[...]
    a2out[...] = h @ wb[...]


def _prob_body(g1, g2, w2, b2, out):
    r = jnp.maximum(g1[...] + g2[...], 0.0)
    out[...] = jax.nn.sigmoid(r @ w2[...] + b2[...])


def _row_spec(cols):
    return pl.BlockSpec((BN, cols), lambda i: (i, 0))


def _full_spec(shape):
    return pl.BlockSpec(shape, lambda i: tuple(0 for _ in shape))


_enc_call = pl.pallas_call(
    _enc_body,
    grid=(N // BN,),
    in_specs=[
        _row_spec(NV),
        _full_spec((NV, H)), _full_spec((1, H)),
        _full_spec((H, H)), _full_spec((1, H)),
        _full_spec((H, H)), _full_spec((1, H)),
        _row_spec(DW), _row_spec(DW),
    ],
    out_specs=_row_spec(H),
    out_shape=jax.ShapeDtypeStruct((N, H), f32),
)

_mid_call = pl.pallas_call(
    _mid_body,
    grid=(N // BN,),
    in_specs=[
        _row_spec(H), _row_spec(H), _row_spec(H),
        _row_spec(DW), _row_spec(DW),
        _full_spec((H, H)), _full_spec((1, H)),
    ],
    out_specs=_row_spec(H),
    out_shape=jax.ShapeDtypeStruct((N, H), f32),
)

_fin_call = pl.pallas_call(
    _fin_body,
    grid=(N // BN,),
    in_specs=[
        _row_spec(H), _row_spec(H), _row_spec(H),
        _row_spec(DW), _row_spec(DW),
        _full_spec((H, H)), _full_spec((H, H)), _full_spec((1, H)),
    ],
    out_specs=[_row_spec(H), _row_spec(H), _row_spec(H)],
    out_shape=[
        jax.ShapeDtypeStruct((N, H), f32),
        jax.ShapeDtypeStruct((N, H), f32),
        jax.ShapeDtypeStruct((N, H), f32),
    ],
)

BE = 4000

_prob_call = pl.pallas_call(
    _prob_body,
    grid=(E // BE,),
    in_specs=[
        pl.BlockSpec((BE, H), lambda i: (i, 0)),
        pl.BlockSpec((BE, H), lambda i: (i, 0)),
        _full_spec((H, 1)), _full_spec((1, 1)),
    ],
    out_specs=pl.BlockSpec((BE, 1), lambda i: (i, 0)),
    out_shape=jax.ShapeDtypeStruct((E, 1), f32),
)


def kernel(x, enc_W1, enc_b1, enc_W2, enc_b2, g1_W, g1_b, g2_W, g2_b,
           ep_W1, ep_b1, ep_W2, ep_b2, edge_index):
    srcr = edge_index[0].reshape(NW, NCH, CH)
    dstr = edge_index[1].reshape(NW, NCH, CH)
    zeros16 = jnp.zeros((NS, RPT, DW), f32)
    zeros64 = jnp.zeros((NS, RPT, H), f32)

    degP = _deg_call()(srcr, zeros16).reshape(NC, N, DW)
    d0, d1 = degP[0], degP[1]

    zs1 = _enc_call(
        x, enc_W1, enc_b1.reshape(1, H), enc_W2, enc_b2.reshape(1, H),
        g1_W, g1_b.reshape(1, H), d0, d1,
    )
    P1 = _msg_call()(srcr, dstr, zs1, zeros64).reshape(NC, N, H)
    zs2 = _mid_call(zs1, P1[0], P1[1], d0, d1, g2_W, g2_b.reshape(1, H))
    P2 = _msg_call()(srcr, dstr, zs2, zeros64).reshape(NC, N, H)
    h, a1, a2 = _fin_call(
        zs2, P2[0], P2[1], d0, d1, ep_W1[:H], ep_W1[H:], ep_b1.reshape(1, H)
    )
    G1, G2 = _edge_call()(srcr, dstr, a1, a2)
    probs = _prob_call(G1.reshape(E, H), G2.reshape(E, H), ep_W2, ep_b2.reshape(1, 1))
    return (probs, h)


# speedup vs baseline: 15.2454x; 3.4953x over previous
import functools

import jax
import jax.numpy as jnp
from jax import lax
from jax.experimental import pallas as pl
from jax.experimental.pallas import tpu as pltpu
from jax.experimental.pallas import tpu_sc as plsc

N = 10000
E = 160000
H = 64
NV = 128
NC, NS = 2, 16
NW = NC * NS
EPW = E // NW
CH = 125
NCH = EPW // CH
RPT = N // NS
DW = 64
NP = N // 2

f32 = jnp.float32


def _sc_mesh():
    return plsc.VectorSubcoreMesh(
        core_axis_name="c", subcore_axis_name="s", num_cores=NC, num_subcores=NS
    )


DWS = 16


def _deg_body(srcr, out, idx_v, ones_v, acc, av, dv, sem):
    c = lax.axis_index("c")
    s = lax.axis_index("s")
    wid = c * NS + s
    pltpu.sync_copy(srcr.at[wid], idx_v)

    @plsc.parallel_loop(0, RPT, 1, unroll=8)
    def _zf(i):
        av[i] = jnp.zeros((16,), f32)

    pltpu.sync_copy(av, acc.at[pl.ds(s * RPT, RPT)])

    @plsc.parallel_loop(0, CH, 1, unroll=8)
    def _of(i):
        ones_v[i] = jnp.full((16,), 1.0, f32)

    plsc.subcore_barrier()

    def chunk(j, carry):
        @pl.when(j >= 8)
        def _():
            pltpu.make_async_copy(ones_v, acc.at[idx_v.at[0]], sem).wait()

        pltpu.async_copy(ones_v, acc.at[idx_v.at[j]], sem, add=True)
        return carry

    lax.fori_loop(0, NCH, chunk, 0)

    def drain(j, carry):
        pltpu.make_async_copy(ones_v, acc.at[idx_v.at[0]], sem).wait()
        return carry

    lax.fori_loop(0, 8, drain, 0)
    plsc.subcore_barrier()
    pltpu.sync_copy(acc.at[pl.ds(s * RPT, RPT)], av)

    @plsc.parallel_loop(0, RPT, 1, unroll=4)
    def _bc(i):
        v = av[i]
        dv[i, pl.ds(0, 16)] = v
        dv[i, pl.ds(16, 16)] = v
        dv[i, pl.ds(32, 16)] = v
        dv[i, pl.ds(48, 16)] = v

    pltpu.sync_copy(dv, out.at[c, s])


@functools.cache
def _deg_call():
    return pl.kernel(
        _deg_body,
        out_type=jax.ShapeDtypeStruct((NC, NS, RPT, DW), f32),
        mesh=_sc_mesh(),
        compiler_params=pltpu.CompilerParams(use_tc_tiling_on_sc=False),
        scratch_types=[
            pltpu.VMEM((NCH, CH), jnp.int32),
            pltpu.VMEM((CH, DWS), f32),
            pltpu.VMEM_SHARED((N, DWS), f32),
            pltpu.VMEM((RPT, DWS), f32),
            pltpu.VMEM((RPT, DW), f32),
            pltpu.SemaphoreType.DMA,
        ],
    )


NSLOT = 8


def _msg_body(srcr, dstr, zs, out, src_v, dst_v, rows, acc, semg, sems):
    c = lax.axis_index("c")
    s = lax.axis_index("s")
    wid = c * NS + s
    pltpu.sync_copy(srcr.at[wid], src_v)
    pltpu.sync_copy(dstr.at[wid], dst_v)

    @plsc.parallel_loop(0, CH, 1, unroll=8)
    def _zf(i):
        for k in range(H // 16):
            rows[0, i, pl.ds(16 * k, 16)] = jnp.zeros((16,), f32)

    for k in range(RPT // CH):
        pltpu.sync_copy(rows.at[0], acc.at[pl.ds(s * RPT + k * CH, CH)])
    plsc.subcore_barrier()
    for j in range(4):
        pltpu.async_copy(zs.at[dst_v.at[j]], rows.at[j], semg[j])

    def octet(jj, carry):
        for i in range(NSLOT):
            j = NSLOT * jj + i
            sn = (i + 4) % NSLOT
            jn = j + 4

            @pl.when(jn < NCH)
            def _(j=j, sn=sn, jn=jn):
                @pl.when(j >= 4)
                def _(sn=sn, j=j):
                    pltpu.make_async_copy(
                        rows.at[sn], acc.at[src_v.at[0]], sems[sn]).wait()

                pltpu.async_copy(zs.at[dst_v.at[jn]], rows.at[sn], semg[sn])

            pltpu.make_async_copy(zs.at[dst_v.at[j]], rows.at[i], semg[i]).wait()
            pltpu.async_copy(rows.at[i], acc.at[src_v.at[j]], sems[i], add=True)
        return carry

    lax.fori_loop(0, NCH // NSLOT, octet, 0)
    for i in range(NSLOT):
        pltpu.make_async_copy(rows.at[i], acc.at[src_v.at[0]], sems[i]).wait()
    plsc.subcore_barrier()
    pltpu.sync_copy(acc.at[pl.ds(s * RPT, RPT)], out.at[c, s])


@functools.cache
def _msg_call():
    return pl.kernel(
        _msg_body,
        out_type=jax.ShapeDtypeStruct((NC, NS, RPT, H), f32),
        mesh=_sc_mesh(),
        compiler_params=pltpu.CompilerParams(use_tc_tiling_on_sc=False),
        scratch_types=[
            pltpu.VMEM((NCH, CH), jnp.int32),
            pltpu.VMEM((NCH, CH), jnp.int32),
            pltpu.VMEM((NSLOT, CH, H), f32),
            pltpu.VMEM_SHARED((N, H), f32),
            [pltpu.SemaphoreType.DMA] * NSLOT,
            [pltpu.SemaphoreType.DMA] * NSLOT,
        ],
    )


PW = 16


def _edge_compute(r1, r2, wv, ob):
    w0 = wv[0]
    w1 = wv[1]
    w2 = wv[2]
    w3 = wv[3]

    @plsc.parallel_loop(0, CH, 1, unroll=5)
    def per_edge(e):
        t0 = jnp.maximum(r1[e, pl.ds(0, 16)] + r2[e, pl.ds(0, 16)], 0.0) * w0
        t1 = jnp.maximum(r1[e, pl.ds(16, 16)] + r2[e, pl.ds(16, 16)], 0.0) * w1
        t2 = jnp.maximum(r1[e, pl.ds(32, 16)] + r2[e, pl.ds(32, 16)], 0.0) * w2
        t3 = jnp.maximum(r1[e, pl.ds(48, 16)] + r2[e, pl.ds(48, 16)], 0.0) * w3
        ob[e] = (t0 + t1) + (t2 + t3)


ESLOT = 5


def _edge_body(srcr, dstr, a1, a2, w2r, out, src_v, dst_v,
               r1, r2, wv, ob, sg, sw):
    c = lax.axis_index("c")
    s = lax.axis_index("s")
    wid = c * NS + s
    pltpu.sync_copy(srcr.at[wid], src_v)
    pltpu.sync_copy(dstr.at[wid], dst_v)
    pltpu.sync_copy(w2r, wv)
    for j in range(3):
        pltpu.async_copy(a1.at[src_v.at[j]], r1.at[j], sg[j])
        pltpu.async_copy(a2.at[dst_v.at[j]], r2.at[j], sg[j])

    def quint(jj, carry):
        for i in range(ESLOT):
            j = ESLOT * jj + i
            sn = (i + 3) % ESLOT
            jn = j + 3

            @pl.when(jn < NCH)
            def _(sn=sn, jn=jn):
                pltpu.async_copy(a1.at[src_v.at[jn]], r1.at[sn], sg[sn])
                pltpu.async_copy(a2.at[dst_v.at[jn]], r2.at[sn], sg[sn])

            pltpu.make_async_copy(a1.at[src_v.at[j]], r1.at[i], sg[i]).wait()
            pltpu.make_async_copy(a2.at[dst_v.at[j]], r2.at[i], sg[i]).wait()

            @pl.when(j >= ESLOT)
            def _(i=i, j=j):
                pltpu.make_async_copy(ob.at[i], out.at[wid, j], sw[i]).wait()

            _edge_compute(r1.at[i], r2.at[i], wv, ob.at[i])
            pltpu.async_copy(ob.at[i], out.at[wid, j], sw[i])
        return carry

    lax.fori_loop(0, NCH // ESLOT, quint, 0)
    for i in range(ESLOT):
        pltpu.make_async_copy(ob.at[i], out.at[wid, 0], sw[i]).wait()


@functools.cache
def _edge_call():
    return pl.kernel(
        _edge_body,
        out_type=jax.ShapeDtypeStruct((NW, NCH, CH, PW), f32),
        mesh=_sc_mesh(),
        compiler_params=pltpu.CompilerParams(use_tc_tiling_on_sc=False),
        scratch_types=[
            pltpu.VMEM((NCH, CH), jnp.int32),
            pltpu.VMEM((NCH, CH), jnp.int32),
            pltpu.VMEM((ESLOT, CH, H), f32),
            pltpu.VMEM((ESLOT, CH, H), f32),
            pltpu.VMEM((4, 16), f32),
            pltpu.VMEM((ESLOT, CH, PW), f32),
            [pltpu.SemaphoreType.DMA] * ESLOT,
            [pltpu.SemaphoreType.DMA] * ESLOT,
        ],
    )


BV = 1000


def _dis(d0, d1):
    return lax.rsqrt(1.0 + d0[0] + d1[0])


def _enc_body(x_ref, w1, b1, w2, b2, gw, gb, d0, d1, zs_ref):
    ne = jnp.maximum(x_ref[...] @ w1[...] + b1[...], 0.0) @ w2[...] + b2[...]
    y = ne @ gw[...] + gb[...]
    zs_ref[...] = _dis(d0, d1) * y


def _mid_body(zs, p0, p1, d0, d1, gw, gb, out):
    dis = _dis(d0, d1)
    h = jnp.maximum(dis * (zs[...] + p0[0] + p1[0]), 0.0)
    out[...] = dis * (h @ gw[...] + gb[...])


def _fin_body(zs, p0, p1, d0, d1, wa, wb, b1, hout, a1out, a2out):
    dis = _dis(d0, d1)
    h = dis * (zs[...] + p0[0] + p1[0])
    hout[...] = h
    a1out[...] = h @ wa[...] + b1[...]
    a2out[...] = h @ wb[...]


def _prob_body(g, b2, out):
    li = lax.broadcasted_iota(jnp.int32, (128, 8), 0)
    ki = lax.broadcasted_iota(jnp.int32, (128, 8), 1)
    sel = (li // PW == ki).astype(f32)
    out[...] = jax.nn.sigmoid(g[...] @ sel + b2[0, 0])


def _row_spec(cols):
    return pl.BlockSpec((BV, cols), lambda i: (i, 0))


def _part_spec(c):
    return pl.BlockSpec((1, BV, 128), lambda i, c=c: (c, i, 0))


def _full_spec(shape):
    return pl.BlockSpec(shape, lambda i: tuple(0 for _ in shape))


_enc_call = pl.pallas_call(
    _enc_body,
    grid=(NP // BV,),
    in_specs=[
        _row_spec(2 * NV),
        _full_spec((2 * NV, 128)), _full_spec((1, 128)),
        _full_spec((128, 128)), _full_spec((1, 128)),
        _full_spec((128, 128)), _full_spec((1, 128)),
        _part_spec(0), _part_spec(1),
    ],
    out_specs=_row_spec(128),
    out_shape=jax.ShapeDtypeStruct((NP, 128), f32),
)

_mid_call = pl.pallas_call(
    _mid_body,
    grid=(NP // BV,),
    in_specs=[
        _row_spec(128), _part_spec(0), _part_spec(1),
        _part_spec(0), _part_spec(1),
        _full_spec((128, 128)), _full_spec((1, 128)),
    ],
    out_specs=_row_spec(128),
    out_shape=jax.ShapeDtypeStruct((NP, 128), f32),
)

_fin_call = pl.pallas_call(
    _fin_body,
    grid=(NP // BV,),
    in_specs=[
        _row_spec(128), _part_spec(0), _part_spec(1),
        _part_spec(0), _part_spec(1),
        _full_spec((128, 128)), _full_spec((128, 128)), _full_spec((1, 128)),
    ],
    out_specs=[_row_spec(128), _row_spec(128), _row_spec(128)],
    out_shape=[
        jax.ShapeDtypeStruct((NP, 128), f32),
        jax.ShapeDtypeStruct((NP, 128), f32),
        jax.ShapeDtypeStruct((NP, 128), f32),
    ],
)

BE = 2000

_prob_call = pl.pallas_call(
    _prob_body,
    grid=(E // 8 // BE,),
    in_specs=[
        pl.BlockSpec((BE, 128), lambda i: (i, 0)),
        _full_spec((1, 1)),
    ],
    out_specs=pl.BlockSpec((BE, 8), lambda i: (i, 0)),
    out_shape=jax.ShapeDtypeStruct((E // 8, 8), f32),
)


def _bd(w):
    fi, fo = w.shape
    z = jnp.zeros((fi, fo), f32)
    return jnp.concatenate(
        [jnp.concatenate([w, z], axis=1), jnp.concatenate([z, w], axis=1)], axis=0
    )


def _b2(b):
    return jnp.concatenate([b, b]).reshape(1, 128)


def kernel(x, enc_W1, enc_b1, enc_W2, enc_b2, g1_W, g1_b, g2_W, g2_b,
           ep_W1, ep_b1, ep_W2, ep_b2, edge_index):
    srcr = edge_index[0].reshape(NW, NCH, CH)
    dstr = edge_index[1].reshape(NW, NCH, CH)
    degP = _deg_call()(srcr).reshape(NC, NP, 128)

    zs1 = _enc_call(
        x.reshape(NP, 2 * NV), _bd(enc_W1), _b2(enc_b1), _bd(enc_W2),
        _b2(enc_b2), _bd(g1_W), _b2(g1_b), degP, degP,
    )
    P1 = _msg_call()(srcr, dstr, zs1.reshape(N, H)).reshape(NC, NP, 128)
    zs2 = _mid_call(zs1, P1, P1, degP, degP, _bd(g2_W), _b2(g2_b))
    P2 = _msg_call()(srcr, dstr, zs2.reshape(N, H)).reshape(NC, NP, 128)
    hv, a1, a2 = _fin_call(
        zs2, P2, P2, degP, degP, _bd(ep_W1[:H]), _bd(ep_W1[H:]), _b2(ep_b1)
    )
    PP = _edge_call()(srcr, dstr, a1.reshape(N, H), a2.reshape(N, H),
                      ep_W2.reshape(4, 16))
    probs = _prob_call(PP.reshape(E // 8, 128), ep_b2.reshape(1, 1)).reshape(E, 1)
    return (probs, hv.reshape(N, H))
